# baseline (speedup 1.0000x reference)
import functools

import jax
import jax.numpy as jnp
from jax import lax
from jax.experimental import pallas as pl
from jax.experimental.pallas import tpu as pltpu
from jax.experimental.pallas import tpu_sc as plsc

N_MODS = 16
EMBED_DIM = 64
BATCH = 16384
N_PLAYERS_Q = 7813
N_MAP_Q = 1250

_info = plsc.get_sparse_core_info()
_NC, _NS, _L = _info.num_cores, _info.num_subcores, _info.num_lanes
_NW = _NC * _NS
_BPW = BATCH // _NW
_NVR = BATCH // _L
_PBLK = (N_PLAYERS_Q + _NW - 1) // _NW
_MBLK = (N_MAP_Q + _NW - 1) // _NW
_NBLK_MAX = _PBLK
_CH = 512
_RING = 8


def _gather_table(tab_t, idx_of, out_rows, nblk, qtot, wid,
                  mr, mb, blk, rowbufs,
                  rp_sm, bp_sm, cnt_sm, off_sm, ne_sm,
                  sem_b0, sem_b1, sem_w):
    iota = lax.iota(jnp.int32, _L)
    q0 = wid * nblk

    def match_body(i, ptr):
        v = idx_of(i)
        q = lax.shift_right_logical(v, 7)
        m = (q >= q0) & (q < q0 + nblk)
        plsc.store_compressed(mr.at[pl.ds(ptr, _L)], v, mask=m)
        plsc.store_compressed(mb.at[pl.ds(ptr, _L)], i * _L + iota, mask=m)
        pc = plsc.all_reduce_population_count(m)
        if getattr(pc, "ndim", 0):
            pc = pc[0]
        return ptr + pc

    m_total = lax.fori_loop(0, _NVR, match_body, jnp.int32(0))

    c16 = [lax.iota(jnp.int32, _L) + k * _L for k in range(EMBED_DIM // _L)]

    def chunk_body(ci, _):
        lo = ci * _CH
        lch = jnp.minimum(jnp.int32(_CH), m_total - lo)

        def zero_body(z, _):
            cnt_sm[z] = 0
            return 0
        lax.fori_loop(0, nblk + 1, zero_body, 0)

        def count_body(j, _):
            bs = j * _L
            rv = mr[pl.ds(lo + bs, _L)]
            for lane in range(_L):
                @pl.when(bs + lane < lch)
                def _():
                    qq = lax.shift_right_logical(rv[lane], 7) - q0
                    cnt_sm[qq] = cnt_sm[qq] + 1
            return 0
        lax.fori_loop(0, _CH // _L, count_body, 0)

        off_sm[0] = 0

        def pfx_body(z, nne):
            c = cnt_sm[z]
            off_sm[z + 1] = off_sm[z] + c
            cnt_sm[z] = off_sm[z]

            @pl.when(c > 0)
            def _():
                ne_sm[nne] = z
            return nne + jnp.where(c > 0, 1, 0)
        nne = lax.fori_loop(0, nblk, pfx_body, jnp.int32(0))

        def place_body(j, _):
            bs = j * _L
            rv = mr[pl.ds(lo + bs, _L)]
            bv = mb[pl.ds(lo + bs, _L)]
            for lane in range(_L):
                @pl.when(bs + lane < lch)
                def _():
                    qq = lax.shift_right_logical(rv[lane], 7) - q0
                    slot = cnt_sm[qq]
                    rp_sm[slot] = rv[lane]
                    bp_sm[slot] = bv[lane]
                    cnt_sm[qq] = slot + 1
            return 0
        lax.fori_loop(0, _CH // _L, place_body, 0)

        @pl.when(nne > 0)
        def _prime():
            pltpu.make_async_copy(tab_t.at[:, pl.ds((q0 + ne_sm[0]) * 128,
                                                    128)],
                                  blk.at[0], sem_b0).start()

        def block_body(t, prev_end):
            par = lax.rem(t, 2)

            @pl.when(t + 1 < nne)
            def _next():
                q1 = q0 + ne_sm[t + 1]

                @pl.when(par == 0)
                def _():
                    pltpu.make_async_copy(tab_t.at[:, pl.ds(q1 * 128, 128)],
                                          blk.at[1], sem_b1).start()

                @pl.when(par == 1)
                def _():
                    pltpu.make_async_copy(tab_t.at[:, pl.ds(q1 * 128, 128)],
                                          blk.at[0], sem_b0).start()

            @pl.when(par == 0)
            def _():
                pltpu.make_async_copy(tab_t.at[:, pl.ds(0, 128)],
                                      blk.at[0], sem_b0).wait()

            @pl.when(par == 1)
            def _():
                pltpu.make_async_copy(tab_t.at[:, pl.ds(0, 128)],
                                      blk.at[1], sem_b1).wait()

            par16 = jnp.full((_L,), par, jnp.int32)
            qq = ne_sm[t]

            def entry_body(e, _):
                @pl.when(e >= _RING)
                def _drain():
                    pltpu.make_async_copy(out_rows.at[0],
                                          rowbufs.at[0], sem_w).wait()
                r = rp_sm[e]
                b = bp_sm[e]
                col = jnp.full((_L,), lax.rem(r, 128), jnp.int32)
                slot = lax.rem(e, _RING)
                for k in range(EMBED_DIM // _L):
                    g = plsc.load_gather(blk, [par16, c16[k], col])
                    rowbufs[slot, pl.ds(k * _L, _L)] = g
                pltpu.make_async_copy(rowbufs.at[slot],
                                      out_rows.at[b], sem_w).start()
                return 0

            ehi = cnt_sm[qq]
            lax.fori_loop(prev_end, ehi, entry_body, 0)
            return ehi

        lax.fori_loop(0, nne, block_body, jnp.int32(0))

        def tail_body(d, _):
            pltpu.make_async_copy(out_rows.at[0], rowbufs.at[0], sem_w).wait()
            return 0
        lax.fori_loop(0, jnp.minimum(jnp.int32(_RING), lch), tail_body, 0)
        return 0

    lax.fori_loop(0, (m_total + _CH - 1) // _CH, chunk_body, 0)


def _gather_body(pidx_hbm, bidx_hbm, mbits_hbm, ptab_t, mtab_t,
                 pg_hbm, mg_hbm,
                 pidx_all, bidx_all, mbits_all, mr, mb, blk, rowbufs,
                 rp_sm, bp_sm, cnt_sm, off_sm, ne_sm,
                 sem_b0, sem_b1, sem_w):
    wid = lax.axis_index("s") * _NC + lax.axis_index("c")

    pltpu.sync_copy(pidx_hbm, pidx_all)
    pltpu.sync_copy(bidx_hbm, bidx_all)
    pltpu.sync_copy(mbits_hbm, mbits_all)

    args = (mr, mb, blk, rowbufs, rp_sm, bp_sm, cnt_sm, off_sm, ne_sm,
            sem_b0, sem_b1, sem_w)

    _gather_table(ptab_t, lambda i: pidx_all[pl.ds(i * _L, _L)],
                  pg_hbm, _PBLK, N_PLAYERS_Q, wid, *args)
    _gather_table(mtab_t,
                  lambda i: (bidx_all[pl.ds(i * _L, _L)] * N_MODS
                             + mbits_all[pl.ds(i * _L, _L)]),
                  mg_hbm, _MBLK, N_MAP_Q, wid, *args)


_RPP = _BPW // 2
_GROUPS = _RPP // _L


def _dot_body(pg_hbm, mg_hbm, out_hbm, prow, mrow, partials, out_v,
              sem_p, sem_m):
    wid = lax.axis_index("s") * _NC + lax.axis_index("c")
    base = wid * _BPW
    iota = lax.iota(jnp.int32, _L)
    colbase = iota * _L

    for p in range(2):
        off = p * _RPP
        cp = pltpu.async_copy(pg_hbm.at[pl.ds(base + off, _RPP)], prow, sem_p)
        cm = pltpu.async_copy(mg_hbm.at[pl.ds(base + off, _RPP)], mrow, sem_m)
        cp.wait()
        cm.wait()

        def group_body(g, _, off=off):
            for r in range(_L):
                rr = g * _L + r
                part = prow[rr, pl.ds(0, _L)] * mrow[rr, pl.ds(0, _L)]
                for k in range(1, EMBED_DIM // _L):
                    sl = pl.ds(k * _L, _L)
                    part = part + prow[rr, sl] * mrow[rr, sl]
                partials[pl.ds(r * _L, _L)] = part
            acc = plsc.load_gather(partials, [colbase])
            for l in range(1, _L):
                acc = acc + plsc.load_gather(partials, [colbase + l])
            out_v[pl.ds(off + g * _L, _L)] = acc
            return 0

        lax.fori_loop(0, _GROUPS, group_body, 0)

    pltpu.sync_copy(out_v, out_hbm.at[pl.ds(base, _BPW)])


@jax.jit
def _run(player_indices, beatmap_ids, mod_bits, ptab_t, mtab_t):
    mesh = plsc.VectorSubcoreMesh(core_axis_name="c", subcore_axis_name="s")
    gather_call = functools.partial(
        pl.kernel,
        out_type=(jax.ShapeDtypeStruct((BATCH, EMBED_DIM), jnp.float32),
                  jax.ShapeDtypeStruct((BATCH, EMBED_DIM), jnp.float32)),
        mesh=mesh,
        compiler_params=pltpu.CompilerParams(needs_layout_passes=False),
        scratch_types=[
            pltpu.VMEM((BATCH,), jnp.int32),
            pltpu.VMEM((BATCH,), jnp.int32),
            pltpu.VMEM((BATCH,), jnp.int32),
            pltpu.VMEM((BATCH + _L,), jnp.int32),
            pltpu.VMEM((BATCH + _L,), jnp.int32),
            pltpu.VMEM((2, EMBED_DIM, 128), jnp.float32),
            pltpu.VMEM((_RING, EMBED_DIM), jnp.float32),
            pltpu.SMEM((_CH,), jnp.int32),
            pltpu.SMEM((_CH,), jnp.int32),
            pltpu.SMEM((_NBLK_MAX + 1,), jnp.int32),
            pltpu.SMEM((_NBLK_MAX + 1,), jnp.int32),
            pltpu.SMEM((_NBLK_MAX + 1,), jnp.int32),
            pltpu.SemaphoreType.DMA,
            pltpu.SemaphoreType.DMA,
            pltpu.SemaphoreType.DMA,
        ],
    )(_gather_body)
    pg, mg = gather_call(player_indices, beatmap_ids, mod_bits,
                         ptab_t, mtab_t)

    dot_call = functools.partial(
        pl.kernel,
        out_type=jax.ShapeDtypeStruct((BATCH,), jnp.float32),
        mesh=mesh,
        compiler_params=pltpu.CompilerParams(needs_layout_passes=False),
        scratch_types=[
            pltpu.VMEM((_RPP, EMBED_DIM), jnp.float32),
            pltpu.VMEM((_RPP, EMBED_DIM), jnp.float32),
            pltpu.VMEM((_L * _L,), jnp.float32),
            pltpu.VMEM((_BPW,), jnp.float32),
            pltpu.SemaphoreType.DMA,
            pltpu.SemaphoreType.DMA,
        ],
    )(_dot_body)
    return dot_call(pg, mg)


def kernel(player_indices, beatmap_ids, mod_bits, player_table, map_table):
    return _run(player_indices.astype(jnp.int32),
                beatmap_ids.astype(jnp.int32),
                mod_bits.astype(jnp.int32),
                player_table.T, map_table.T)


# speedup vs baseline: 1.1169x; 1.1169x over previous
import functools

import jax
import jax.numpy as jnp
from jax import lax
from jax.experimental import pallas as pl
from jax.experimental.pallas import tpu as pltpu
from jax.experimental.pallas import tpu_sc as plsc

N_MODS = 16
EMBED_DIM = 64
BATCH = 16384
N_PLAYERS_Q = 7813
N_MAP_Q = 1250

_info = plsc.get_sparse_core_info()
_NC, _NS, _L = _info.num_cores, _info.num_subcores, _info.num_lanes
_NW = _NC * _NS
_BPW = BATCH // _NW
_NVR = BATCH // _L
_PBLK = (N_PLAYERS_Q + _NW - 1) // _NW
_MBLK = (N_MAP_Q + _NW - 1) // _NW
_NBLK_MAX = _PBLK
_CH = 576
_RING = 8


def _gather_table(tab_t, idx_of, out_rows, nblk, qtot, wid,
                  mr, mb, blk, rowbufs,
                  rp_sm, bp_sm, cnt_sm, off_sm,
                  sem_b0, sem_b1, sem_w):
    iota = lax.iota(jnp.int32, _L)
    q0 = wid * nblk

    def match_body(i, ptr):
        v = idx_of(i)
        q = lax.shift_right_logical(v, 7)
        m = (q >= q0) & (q < q0 + nblk)
        plsc.store_compressed(mr.at[pl.ds(ptr, _L)], v, mask=m)
        plsc.store_compressed(mb.at[pl.ds(ptr, _L)], i * _L + iota, mask=m)
        pc = plsc.all_reduce_population_count(m)
        if getattr(pc, "ndim", 0):
            pc = pc[0]
        return ptr + pc

    m_total = lax.fori_loop(0, _NVR, match_body, jnp.int32(0))

    c16 = [lax.iota(jnp.int32, _L) + k * _L for k in range(EMBED_DIM // _L)]

    def chunk_body(ci, _):
        lo = ci * _CH
        lch = jnp.minimum(jnp.int32(_CH), m_total - lo)

        def zero_body(z, _):
            cnt_sm[z] = 0
            return 0
        lax.fori_loop(0, nblk + 1, zero_body, 0)

        def count_body(j, _):
            bs = j * _L
            rv = mr[pl.ds(lo + bs, _L)]
            for lane in range(_L):
                @pl.when(bs + lane < lch)
                def _():
                    qq = lax.shift_right_logical(rv[lane], 7) - q0
                    cnt_sm[qq] = cnt_sm[qq] + 1
            return 0
        lax.fori_loop(0, _CH // _L, count_body, 0)

        off_sm[0] = 0

        def pfx_body(z, nne):
            c = cnt_sm[z]
            oz = off_sm[z]
            off_sm[z + 1] = oz + c
            cnt_sm[z] = oz

            @pl.when(c > 0)
            def _():
                off_sm[nne] = z
            return nne + jnp.where(c > 0, 1, 0)
        nne = lax.fori_loop(0, nblk, pfx_body, jnp.int32(0))
        ne_sm = off_sm

        def place_body(j, _):
            bs = j * _L
            rv = mr[pl.ds(lo + bs, _L)]
            bv = mb[pl.ds(lo + bs, _L)]
            for lane in range(_L):
                @pl.when(bs + lane < lch)
                def _():
                    qq = lax.shift_right_logical(rv[lane], 7) - q0
                    slot = cnt_sm[qq]
                    rp_sm[slot] = rv[lane]
                    bp_sm[slot] = bv[lane]
                    cnt_sm[qq] = slot + 1
            return 0
        lax.fori_loop(0, _CH // _L, place_body, 0)

        @pl.when(nne > 0)
        def _prime():
            pltpu.make_async_copy(tab_t.at[:, pl.ds((q0 + ne_sm[0]) * 128,
                                                    128)],
                                  blk.at[0], sem_b0).start()

        def block_body(t, prev_end):
            par = lax.rem(t, 2)

            @pl.when(t + 1 < nne)
            def _next():
                q1 = q0 + ne_sm[t + 1]

                @pl.when(par == 0)
                def _():
                    pltpu.make_async_copy(tab_t.at[:, pl.ds(q1 * 128, 128)],
                                          blk.at[1], sem_b1).start()

                @pl.when(par == 1)
                def _():
                    pltpu.make_async_copy(tab_t.at[:, pl.ds(q1 * 128, 128)],
                                          blk.at[0], sem_b0).start()

            @pl.when(par == 0)
            def _():
                pltpu.make_async_copy(tab_t.at[:, pl.ds(0, 128)],
                                      blk.at[0], sem_b0).wait()

            @pl.when(par == 1)
            def _():
                pltpu.make_async_copy(tab_t.at[:, pl.ds(0, 128)],
                                      blk.at[1], sem_b1).wait()

            par16 = jnp.full((_L,), par, jnp.int32)
            qq = ne_sm[t]

            def entry_body(e, _):
                @pl.when(e >= _RING)
                def _drain():
                    pltpu.make_async_copy(out_rows.at[0],
                                          rowbufs.at[0], sem_w).wait()
                r = rp_sm[e]
                b = bp_sm[e]
                col = jnp.full((_L,), lax.rem(r, 128), jnp.int32)
                slot = lax.rem(e, _RING)
                for k in range(EMBED_DIM // _L):
                    g = plsc.load_gather(blk, [par16, c16[k], col])
                    rowbufs[slot, pl.ds(k * _L, _L)] = g
                pltpu.make_async_copy(rowbufs.at[slot],
                                      out_rows.at[b], sem_w).start()
                return 0

            ehi = cnt_sm[qq]
            lax.fori_loop(prev_end, ehi, entry_body, 0)
            return ehi

        lax.fori_loop(0, nne, block_body, jnp.int32(0))

        def tail_body(d, _):
            pltpu.make_async_copy(out_rows.at[0], rowbufs.at[0], sem_w).wait()
            return 0
        lax.fori_loop(0, jnp.minimum(jnp.int32(_RING), lch), tail_body, 0)
        return 0

    lax.fori_loop(0, (m_total + _CH - 1) // _CH, chunk_body, 0)


def _gather_body(pidx_hbm, bidx_hbm, mbits_hbm, ptab_t, mtab_t,
                 pg_hbm, mg_hbm,
                 pidx_all, bidx_all, mbits_all, mr, mb, blk, rowbufs,
                 rp_sm, bp_sm, cnt_sm, off_sm,
                 sem_b0, sem_b1, sem_w):
    wid = lax.axis_index("s") * _NC + lax.axis_index("c")

    pltpu.sync_copy(pidx_hbm, pidx_all)
    pltpu.sync_copy(bidx_hbm, bidx_all)
    pltpu.sync_copy(mbits_hbm, mbits_all)

    args = (mr, mb, blk, rowbufs, rp_sm, bp_sm, cnt_sm, off_sm,
            sem_b0, sem_b1, sem_w)

    _gather_table(ptab_t, lambda i: pidx_all[pl.ds(i * _L, _L)],
                  pg_hbm, _PBLK, N_PLAYERS_Q, wid, *args)
    _gather_table(mtab_t,
                  lambda i: (bidx_all[pl.ds(i * _L, _L)] * N_MODS
                             + mbits_all[pl.ds(i * _L, _L)]),
                  mg_hbm, _MBLK, N_MAP_Q, wid, *args)


_RPP = _BPW // 2
_GROUPS = _RPP // _L


def _dot_body(pg_hbm, mg_hbm, out_hbm, prow, mrow, partials, out_v,
              sem_p, sem_m):
    wid = lax.axis_index("s") * _NC + lax.axis_index("c")
    base = wid * _BPW
    iota = lax.iota(jnp.int32, _L)
    colbase = iota * _L

    for p in range(2):
        off = p * _RPP
        cp = pltpu.async_copy(pg_hbm.at[pl.ds(base + off, _RPP)], prow, sem_p)
        cm = pltpu.async_copy(mg_hbm.at[pl.ds(base + off, _RPP)], mrow, sem_m)
        cp.wait()
        cm.wait()

        def group_body(g, _, off=off):
            for r in range(_L):
                rr = g * _L + r
                part = prow[rr, pl.ds(0, _L)] * mrow[rr, pl.ds(0, _L)]
                for k in range(1, EMBED_DIM // _L):
                    sl = pl.ds(k * _L, _L)
                    part = part + prow[rr, sl] * mrow[rr, sl]
                partials[pl.ds(r * _L, _L)] = part
            acc = plsc.load_gather(partials, [colbase])
            for l in range(1, _L):
                acc = acc + plsc.load_gather(partials, [colbase + l])
            out_v[pl.ds(off + g * _L, _L)] = acc
            return 0

        lax.fori_loop(0, _GROUPS, group_body, 0)

    pltpu.sync_copy(out_v, out_hbm.at[pl.ds(base, _BPW)])


@jax.jit
def _run(player_indices, beatmap_ids, mod_bits, ptab_t, mtab_t):
    mesh = plsc.VectorSubcoreMesh(core_axis_name="c", subcore_axis_name="s")
    gather_call = functools.partial(
        pl.kernel,
        out_type=(jax.ShapeDtypeStruct((BATCH, EMBED_DIM), jnp.float32),
                  jax.ShapeDtypeStruct((BATCH, EMBED_DIM), jnp.float32)),
        mesh=mesh,
        compiler_params=pltpu.CompilerParams(needs_layout_passes=False),
        scratch_types=[
            pltpu.VMEM((BATCH,), jnp.int32),
            pltpu.VMEM((BATCH,), jnp.int32),
            pltpu.VMEM((BATCH,), jnp.int32),
            pltpu.VMEM((BATCH + _L,), jnp.int32),
            pltpu.VMEM((BATCH + _L,), jnp.int32),
            pltpu.VMEM((2, EMBED_DIM, 128), jnp.float32),
            pltpu.VMEM((_RING, EMBED_DIM), jnp.float32),
            pltpu.SMEM((_CH,), jnp.int32),
            pltpu.SMEM((_CH,), jnp.int32),
            pltpu.SMEM((_NBLK_MAX + 1,), jnp.int32),
            pltpu.SMEM((_NBLK_MAX + 1,), jnp.int32),
            pltpu.SemaphoreType.DMA,
            pltpu.SemaphoreType.DMA,
            pltpu.SemaphoreType.DMA,
        ],
    )(_gather_body)
    pg, mg = gather_call(player_indices, beatmap_ids, mod_bits,
                         ptab_t, mtab_t)

    dot_call = functools.partial(
        pl.kernel,
        out_type=jax.ShapeDtypeStruct((BATCH,), jnp.float32),
        mesh=mesh,
        compiler_params=pltpu.CompilerParams(needs_layout_passes=False),
        scratch_types=[
            pltpu.VMEM((_RPP, EMBED_DIM), jnp.float32),
            pltpu.VMEM((_RPP, EMBED_DIM), jnp.float32),
            pltpu.VMEM((_L * _L,), jnp.float32),
            pltpu.VMEM((_BPW,), jnp.float32),
            pltpu.SemaphoreType.DMA,
            pltpu.SemaphoreType.DMA,
        ],
    )(_dot_body)
    return dot_call(pg, mg)


def kernel(player_indices, beatmap_ids, mod_bits, player_table, map_table):
    return _run(player_indices.astype(jnp.int32),
                beatmap_ids.astype(jnp.int32),
                mod_bits.astype(jnp.int32),
                player_table.T, map_table.T)


# speedup vs baseline: 1.4126x; 1.2648x over previous
import functools

import jax
import jax.numpy as jnp
from jax import lax
from jax.experimental import pallas as pl
from jax.experimental.pallas import tpu as pltpu
from jax.experimental.pallas import tpu_sc as plsc

N_MODS = 16
EMBED_DIM = 64
BATCH = 16384
N_PLAYERS_Q = 7813
N_MAP_Q = 1250

_info = plsc.get_sparse_core_info()
_NC, _NS, _L = _info.num_cores, _info.num_subcores, _info.num_lanes
_NW = _NC * _NS
_BPW = BATCH // _NW
_NVR = BATCH // _L
_PBLK = (N_PLAYERS_Q + _NW - 1) // _NW
_MBLK = (N_MAP_Q + _NW - 1) // _NW
_NBLK_MAX = _PBLK
_CH = 576
_RING = 8
_BDEPTH = 4


def _gather_table(tab_t, idx_of, out_rows, nblk, qtot, wid,
                  mr, mb, blk, rowbufs,
                  rp_sm, bp_sm, cnt_sm, off_sm,
                  sem_b0, sem_b1, sem_b2, sem_b3, sem_w):
    iota = lax.iota(jnp.int32, _L)
    q0 = wid * nblk

    def match_body(i, ptr):
        v = idx_of(i)
        q = lax.shift_right_logical(v, 7)
        m = (q >= q0) & (q < q0 + nblk)
        plsc.store_compressed(mr.at[pl.ds(ptr, _L)], v, mask=m)
        plsc.store_compressed(mb.at[pl.ds(ptr, _L)], i * _L + iota, mask=m)
        pc = plsc.all_reduce_population_count(m)
        if getattr(pc, "ndim", 0):
            pc = pc[0]
        return ptr + pc

    m_total = lax.fori_loop(0, _NVR, match_body, jnp.int32(0))

    c16 = [lax.iota(jnp.int32, _L) + k * _L for k in range(EMBED_DIM // _L)]

    def chunk_body(ci, _):
        lo = ci * _CH
        lch = jnp.minimum(jnp.int32(_CH), m_total - lo)

        def zero_body(z, _):
            cnt_sm[z] = 0
            return 0
        lax.fori_loop(0, nblk + 1, zero_body, 0)

        def count_body(j, _):
            bs = j * _L
            rv = mr[pl.ds(lo + bs, _L)]
            for lane in range(_L):
                @pl.when(bs + lane < lch)
                def _():
                    qq = lax.shift_right_logical(rv[lane], 7) - q0
                    cnt_sm[qq] = cnt_sm[qq] + 1
            return 0
        lax.fori_loop(0, _CH // _L, count_body, 0)

        off_sm[0] = 0

        def pfx_body(z, nne):
            c = cnt_sm[z]
            oz = off_sm[z]
            off_sm[z + 1] = oz + c
            cnt_sm[z] = oz

            @pl.when(c > 0)
            def _():
                off_sm[nne] = z
            return nne + jnp.where(c > 0, 1, 0)
        nne = lax.fori_loop(0, nblk, pfx_body, jnp.int32(0))
        ne_sm = off_sm

        def place_body(j, _):
            bs = j * _L
            rv = mr[pl.ds(lo + bs, _L)]
            bv = mb[pl.ds(lo + bs, _L)]
            for lane in range(_L):
                @pl.when(bs + lane < lch)
                def _():
                    qq = lax.shift_right_logical(rv[lane], 7) - q0
                    slot = cnt_sm[qq]
                    rp_sm[slot] = rv[lane]
                    bp_sm[slot] = bv[lane]
                    cnt_sm[qq] = slot + 1
            return 0
        lax.fori_loop(0, _CH // _L, place_body, 0)

        sems = [sem_b0, sem_b1, sem_b2, sem_b3]
        for d in range(_BDEPTH - 1):
            @pl.when(nne > d)
            def _prime(d=d):
                pltpu.make_async_copy(
                    tab_t.at[:, pl.ds((q0 + ne_sm[d]) * 128, 128)],
                    blk.at[d], sems[d]).start()

        def block_body(t, prev_end):
            par = lax.rem(t, _BDEPTH)
            tn = t + _BDEPTH - 1

            @pl.when(tn < nne)
            def _next():
                q1 = q0 + ne_sm[tn]
                parn = lax.rem(tn, _BDEPTH)
                for d in range(_BDEPTH):
                    @pl.when(parn == d)
                    def _(d=d):
                        pltpu.make_async_copy(
                            tab_t.at[:, pl.ds(q1 * 128, 128)],
                            blk.at[d], sems[d]).start()

            for d in range(_BDEPTH):
                @pl.when(par == d)
                def _(d=d):
                    pltpu.make_async_copy(tab_t.at[:, pl.ds(0, 128)],
                                          blk.at[d], sems[d]).wait()

            par16 = jnp.full((_L,), par, jnp.int32)
            qq = ne_sm[t]

            def entry_body(e, _):
                @pl.when(e >= _RING)
                def _drain():
                    pltpu.make_async_copy(out_rows.at[0],
                                          rowbufs.at[0], sem_w).wait()
                r = rp_sm[e]
                b = bp_sm[e]
                col = jnp.full((_L,), lax.rem(r, 128), jnp.int32)
                slot = lax.rem(e, _RING)
                for k in range(EMBED_DIM // _L):
                    g = plsc.load_gather(blk, [par16, c16[k], col])
                    rowbufs[slot, pl.ds(k * _L, _L)] = g
                pltpu.make_async_copy(rowbufs.at[slot],
                                      out_rows.at[b], sem_w).start()
                return 0

            ehi = cnt_sm[qq]
            lax.fori_loop(prev_end, ehi, entry_body, 0)
            return ehi

        lax.fori_loop(0, nne, block_body, jnp.int32(0))

        def tail_body(d, _):
            pltpu.make_async_copy(out_rows.at[0], rowbufs.at[0], sem_w).wait()
            return 0
        lax.fori_loop(0, jnp.minimum(jnp.int32(_RING), lch), tail_body, 0)
        return 0

    lax.fori_loop(0, (m_total + _CH - 1) // _CH, chunk_body, 0)


def _gather_body(pidx_hbm, bidx_hbm, mbits_hbm, ptab_t, mtab_t,
                 pg_hbm, mg_hbm,
                 pidx_all, bidx_all, mbits_all, mr, mb, blk, rowbufs,
                 rp_sm, bp_sm, cnt_sm, off_sm,
                 sem_b0, sem_b1, sem_b2, sem_b3, sem_w):
    wid = lax.axis_index("s") * _NC + lax.axis_index("c")

    pltpu.sync_copy(pidx_hbm, pidx_all)
    pltpu.sync_copy(bidx_hbm, bidx_all)
    pltpu.sync_copy(mbits_hbm, mbits_all)

    args = (mr, mb, blk, rowbufs, rp_sm, bp_sm, cnt_sm, off_sm,
            sem_b0, sem_b1, sem_b2, sem_b3, sem_w)

    _gather_table(ptab_t, lambda i: pidx_all[pl.ds(i * _L, _L)],
                  pg_hbm, _PBLK, N_PLAYERS_Q, wid, *args)
    _gather_table(mtab_t,
                  lambda i: (bidx_all[pl.ds(i * _L, _L)] * N_MODS
                             + mbits_all[pl.ds(i * _L, _L)]),
                  mg_hbm, _MBLK, N_MAP_Q, wid, *args)


_RPP = _BPW // 2
_GROUPS = _RPP // _L


def _dot_body(pg_hbm, mg_hbm, out_hbm, prow, mrow, partials, out_v,
              sem_p, sem_m):
    wid = lax.axis_index("s") * _NC + lax.axis_index("c")
    base = wid * _BPW
    iota = lax.iota(jnp.int32, _L)
    colbase = iota * _L

    for p in range(2):
        off = p * _RPP
        cp = pltpu.async_copy(pg_hbm.at[pl.ds(base + off, _RPP)], prow, sem_p)
        cm = pltpu.async_copy(mg_hbm.at[pl.ds(base + off, _RPP)], mrow, sem_m)
        cp.wait()
        cm.wait()

        def group_body(g, _, off=off):
            for r in range(_L):
                rr = g * _L + r
                part = prow[rr, pl.ds(0, _L)] * mrow[rr, pl.ds(0, _L)]
                for k in range(1, EMBED_DIM // _L):
                    sl = pl.ds(k * _L, _L)
                    part = part + prow[rr, sl] * mrow[rr, sl]
                partials[pl.ds(r * _L, _L)] = part
            acc = plsc.load_gather(partials, [colbase])
            for l in range(1, _L):
                acc = acc + plsc.load_gather(partials, [colbase + l])
            out_v[pl.ds(off + g * _L, _L)] = acc
            return 0

        lax.fori_loop(0, _GROUPS, group_body, 0)

    pltpu.sync_copy(out_v, out_hbm.at[pl.ds(base, _BPW)])


@jax.jit
def _run(player_indices, beatmap_ids, mod_bits, ptab_t, mtab_t):
    mesh = plsc.VectorSubcoreMesh(core_axis_name="c", subcore_axis_name="s")
    gather_call = functools.partial(
        pl.kernel,
        out_type=(jax.ShapeDtypeStruct((BATCH, EMBED_DIM), jnp.float32),
                  jax.ShapeDtypeStruct((BATCH, EMBED_DIM), jnp.float32)),
        mesh=mesh,
        compiler_params=pltpu.CompilerParams(needs_layout_passes=False),
        scratch_types=[
            pltpu.VMEM((BATCH,), jnp.int32),
            pltpu.VMEM((BATCH,), jnp.int32),
            pltpu.VMEM((BATCH,), jnp.int32),
            pltpu.VMEM((BATCH + _L,), jnp.int32),
            pltpu.VMEM((BATCH + _L,), jnp.int32),
            pltpu.VMEM((_BDEPTH, EMBED_DIM, 128), jnp.float32),
            pltpu.VMEM((_RING, EMBED_DIM), jnp.float32),
            pltpu.SMEM((_CH,), jnp.int32),
            pltpu.SMEM((_CH,), jnp.int32),
            pltpu.SMEM((_NBLK_MAX + 1,), jnp.int32),
            pltpu.SMEM((_NBLK_MAX + 1,), jnp.int32),
            pltpu.SemaphoreType.DMA,
            pltpu.SemaphoreType.DMA,
            pltpu.SemaphoreType.DMA,
            pltpu.SemaphoreType.DMA,
            pltpu.SemaphoreType.DMA,
        ],
    )(_gather_body)
    pg, mg = gather_call(player_indices, beatmap_ids, mod_bits,
                         ptab_t, mtab_t)

    dot_call = functools.partial(
        pl.kernel,
        out_type=jax.ShapeDtypeStruct((BATCH,), jnp.float32),
        mesh=mesh,
        compiler_params=pltpu.CompilerParams(needs_layout_passes=False),
        scratch_types=[
            pltpu.VMEM((_RPP, EMBED_DIM), jnp.float32),
            pltpu.VMEM((_RPP, EMBED_DIM), jnp.float32),
            pltpu.VMEM((_L * _L,), jnp.float32),
            pltpu.VMEM((_BPW,), jnp.float32),
            pltpu.SemaphoreType.DMA,
            pltpu.SemaphoreType.DMA,
        ],
    )(_dot_body)
    return dot_call(pg, mg)


def kernel(player_indices, beatmap_ids, mod_bits, player_table, map_table):
    return _run(player_indices.astype(jnp.int32),
                beatmap_ids.astype(jnp.int32),
                mod_bits.astype(jnp.int32),
                player_table.T, map_table.T)
